# baseline (speedup 1.0000x reference)
import functools

import jax
import jax.numpy as jnp
from jax import lax
from jax.experimental import pallas as pl
from jax.experimental.pallas import tpu as pltpu
from jax.experimental.pallas import tpu_sc as plsc

B, S, H = 4, 2048, 1024
T = B * S
NR = 64
NE = NR + 1
EP = 128
IM = 512
ISH = 1024
UNCOND = 1000
ECNT = 72

TS = 1024
NSTEP = T // TS

TILE = 512
TILE_SH = 9
NT_G = (T * 2 + NE * (TILE - 1) + TILE - 1) // TILE
NP = NT_G * TILE
TE_LEN = 128

NW = 32
TPW = T // NW
CH = 32
NCH = TPW // CH
CCH = 8
NCC = TPW // CCH


def _router_body(uncond_ref, x_ref, cc_ref,
                 e0_ref, e1_ref, w0_ref, w1_ref, r0_ref, r1_ref, counts_ref,
                 x16_ref, tri16):
    step = pl.program_id(0)
    x = x_ref[...]
    cc = cc_ref[...]
    xn = x * (1.0 / jnp.maximum(
        jnp.sqrt(jnp.sum(x * x, axis=1, keepdims=True)), 1e-12))
    cn = cc * (1.0 / jnp.maximum(
        jnp.sqrt(jnp.sum(cc * cc, axis=1, keepdims=True)), 1e-12))
    cos = lax.dot_general(cn, xn, (((1,), (1,)), ((), ())),
                          preferred_element_type=jnp.float32)
    mx = jnp.max(cos, axis=0, keepdims=True)
    pexp = jnp.exp(cos - mx)
    w = pexp / jnp.sum(pexp, axis=0, keepdims=True)

    eio = lax.broadcasted_iota(jnp.int32, (NR, TS), 0).astype(jnp.float32)
    m1 = jnp.max(w, axis=0, keepdims=True)
    a1 = jnp.min(jnp.where(w == m1, eio, jnp.float32(EP)), axis=0, keepdims=True)
    wm = jnp.where(eio == a1, -1.0, w)
    m2 = jnp.max(wm, axis=0, keepdims=True)
    a2 = jnp.min(jnp.where(wm == m2, eio, jnp.float32(EP)), axis=0, keepdims=True)

    is_u = uncond_ref[0] > 0.5
    e0 = jnp.where(is_u, jnp.float32(NE - 1), a1)
    e1 = jnp.where(is_u, jnp.float32(NE - 1), a2)
    w0 = jnp.where(is_u, 1.0, m1)
    w1 = jnp.where(is_u, 0.0, m2)

    ei72 = lax.broadcasted_iota(jnp.int32, (ECNT, TS), 0).astype(jnp.float32)
    oh0 = (ei72 == e0).astype(jnp.bfloat16)
    oh1 = (ei72 == e1).astype(jnp.bfloat16)

    @pl.when(step == 0)
    def _():
        tri16[...] = (lax.broadcasted_iota(jnp.int32, (TS, TS), 0) <
                      lax.broadcasted_iota(jnp.int32, (TS, TS), 1)
                      ).astype(jnp.bfloat16)
        counts_ref[...] = jnp.zeros_like(counts_ref)

    tri = tri16[...]
    p0 = lax.dot_general(oh0, tri, (((1,), (0,)), ((), ())),
                         preferred_element_type=jnp.float32)
    p1 = lax.dot_general(oh1, tri, (((1,), (0,)), ((), ())),
                         preferred_element_type=jnp.float32)
    oh0f = oh0.astype(jnp.float32)
    oh1f = oh1.astype(jnp.float32)
    c0 = jnp.sum(oh0f, axis=1, keepdims=True)
    c1 = jnp.sum(oh1f, axis=1, keepdims=True)

    run = counts_ref[0:ECNT, 0:1]
    r0 = jnp.sum(oh0f * (run + p0), axis=0, keepdims=True)
    r1 = jnp.sum(oh1f * (run + c0 + p1), axis=0, keepdims=True)
    counts_ref[0:ECNT, :] = jnp.broadcast_to(run + c0 + c1, (ECNT, EP))

    e0_ref[0] = e0.astype(jnp.int32)
    e1_ref[0] = e1.astype(jnp.int32)
    w0_ref[0] = w0
    w1_ref[0] = w1
    r0_ref[0] = r0.astype(jnp.int32)
    r1_ref[0] = r1.astype(jnp.int32)
    x16_ref[...] = x.astype(jnp.bfloat16)


def _run_router(flat, uncond, cc):
    i1 = jax.ShapeDtypeStruct((NSTEP, 1, TS), jnp.int32)
    f1 = jax.ShapeDtypeStruct((NSTEP, 1, TS), jnp.float32)
    out_shape = (i1, i1, f1, f1, i1, i1,
                 jax.ShapeDtypeStruct((EP, EP), jnp.float32),
                 jax.ShapeDtypeStruct((T, H), jnp.bfloat16))
    vec_spec = pl.BlockSpec((1, 1, TS), lambda i: (i, 0, 0))
    return pl.pallas_call(
        _router_body,
        grid=(NSTEP,),
        in_specs=[vec_spec,
                  pl.BlockSpec((TS, H), lambda i: (i, 0)),
                  pl.BlockSpec((NR, H), lambda i: (0, 0))],
        out_specs=(vec_spec, vec_spec, vec_spec, vec_spec, vec_spec, vec_spec,
                   pl.BlockSpec((EP, EP), lambda i: (0, 0)),
                   pl.BlockSpec((TS, H), lambda i: (i, 0))),
        out_shape=out_shape,
        scratch_shapes=[pltpu.VMEM((TS, TS), jnp.bfloat16)],
    )(uncond, flat, cc)


def _dispatch_body(counts_hbm, e0_hbm, e1_hbm, r0_hbm, r1_hbm, flat_hbm,
                   xs_hbm, d0_hbm, d1_hbm, te_hbm,
                   counts_v, pend_v, pstart_v, te_v, ev, rv, d0_v, d1_v,
                   rows_v, rsem, ssem0, ssem1):
    wid = lax.axis_index("s") * 2 + lax.axis_index("c")
    base = wid * TPW

    pltpu.sync_copy(counts_hbm, counts_v)
    carry = jnp.int32(0)
    for c in range(EP // 16):
        sl = pl.ds(c * 16, 16)
        cv = counts_v[sl]
        pad = ((cv + (TILE - 1)) >> TILE_SH) << TILE_SH
        cs = plsc.cumsum(pad) + carry
        pend_v[sl] = cs
        pstart_v[sl] = cs - pad
        carry = jnp.max(cs)

    for c in range(TE_LEN // 16):
        thresh = (lax.iota(jnp.int32, 16) + c * 16) * TILE

        def _cnt(e, acc):
            pe = pend_v[pl.ds(e, 16)][0]
            return acc + jnp.where(pe <= thresh, 1, 0)

        cnt = lax.fori_loop(0, NE, _cnt, jnp.zeros((16,), jnp.int32))
        te_v[pl.ds(c * 16, 16)] = jnp.minimum(cnt, NE - 1)
    te_v[pl.ds(112, 16)] = jnp.zeros((16,), jnp.int32) + (carry >> TILE_SH)

    @pl.when(wid == 0)
    def _():
        pltpu.sync_copy(te_v, te_hbm)

    for e_hbm, r_hbm, d_v, d_hbm in ((e0_hbm, r0_hbm, d0_v, d0_hbm),
                                     (e1_hbm, r1_hbm, d1_v, d1_hbm)):
        pltpu.sync_copy(e_hbm.at[pl.ds(base, TPW)], ev)
        pltpu.sync_copy(r_hbm.at[pl.ds(base, TPW)], rv)
        for c in range(TPW // 16):
            sl = pl.ds(c * 16, 16)
            ps = plsc.load_gather(pstart_v, [ev[sl]])
            d_v[c * 16 // CH, pl.ds(c * 16 % CH, 16)] = ps + rv[sl]
        pltpu.sync_copy(d_v, d_hbm.at[wid])

    ssem = (ssem0, ssem1)
    rd = [None] * NCH
    sc = [None] * (2 * NCH)

    def _read(j):
        return pltpu.async_copy(
            flat_hbm.at[pl.ds(base + j * CH, CH)], rows_v.at[j % 2], rsem)

    rd[0] = _read(0)
    for j in range(NCH):
        rd[j].wait()
        sc[2 * j] = pltpu.async_copy(
            rows_v.at[j % 2], xs_hbm.at[d0_v.at[j]], ssem[j % 2])
        sc[2 * j + 1] = pltpu.async_copy(
            rows_v.at[j % 2], xs_hbm.at[d1_v.at[j]], ssem[j % 2])
        if j + 1 < NCH:
            if j >= 1:
                sc[2 * (j - 1)].wait()
                sc[2 * (j - 1) + 1].wait()
            rd[j + 1] = _read(j + 1)
    sc[-2].wait()
    sc[-1].wait()


def _run_dispatch(counts_i, e0f, e1f, r0f, r1f, flat):
    mesh = plsc.VectorSubcoreMesh(core_axis_name="c", subcore_axis_name="s")
    out_type = (jax.ShapeDtypeStruct((NP, H // 2), jnp.int32),
                jax.ShapeDtypeStruct((NW, NCH, CH), jnp.int32),
                jax.ShapeDtypeStruct((NW, NCH, CH), jnp.int32),
                jax.ShapeDtypeStruct((TE_LEN,), jnp.int32))
    scratch = [pltpu.VMEM((EP,), jnp.int32),
               pltpu.VMEM((EP,), jnp.int32),
               pltpu.VMEM((EP,), jnp.int32),
               pltpu.VMEM((TE_LEN,), jnp.int32),
               pltpu.VMEM((TPW,), jnp.int32),
               pltpu.VMEM((TPW,), jnp.int32),
               pltpu.VMEM((NCH, CH), jnp.int32),
               pltpu.VMEM((NCH, CH), jnp.int32),
               pltpu.VMEM((2, CH, H // 2), jnp.int32),
               pltpu.SemaphoreType.DMA,
               pltpu.SemaphoreType.DMA,
               pltpu.SemaphoreType.DMA]
    fn = pl.kernel(_dispatch_body, out_type=out_type, mesh=mesh,
                   scratch_types=scratch,
                   compiler_params=pltpu.CompilerParams(needs_layout_passes=False))
    return fn(counts_i, e0f, e1f, r0f, r1f, flat)


def _silu(g):
    return g * (1.0 / (1.0 + jnp.exp(-g)))


def _gmlp_body(te_ref, x_ref, wg_ref, wu_ref, wd_ref, y_ref):
    @pl.when(pl.program_id(0) < te_ref[112])
    def _():
        x = x_ref[...]
        wg = wg_ref[0].astype(jnp.bfloat16)
        wu = wu_ref[0].astype(jnp.bfloat16)
        wd = wd_ref[0].astype(jnp.bfloat16)
        g = jnp.dot(x, wg, preferred_element_type=jnp.float32)
        up = jnp.dot(x, wu, preferred_element_type=jnp.float32)
        h = (_silu(g) * up).astype(jnp.bfloat16)
        y_ref[...] = jnp.dot(h, wd, preferred_element_type=jnp.float32)


def _run_gmlp(te, xs, Wg, Wu, Wd):
    grid_spec = pltpu.PrefetchScalarGridSpec(
        num_scalar_prefetch=1,
        grid=(NT_G,),
        in_specs=[pl.BlockSpec((TILE, H),
                               lambda i, te: (jnp.minimum(i, te[112] - 1), 0)),
                  pl.BlockSpec((1, H, IM),
                               lambda i, te: (te[jnp.minimum(i, te[112] - 1)], 0, 0)),
                  pl.BlockSpec((1, H, IM),
                               lambda i, te: (te[jnp.minimum(i, te[112] - 1)], 0, 0)),
                  pl.BlockSpec((1, IM, H),
                               lambda i, te: (te[jnp.minimum(i, te[112] - 1)], 0, 0))],
        out_specs=pl.BlockSpec((TILE, H), lambda i, te: (i, 0)),
    )
    return pl.pallas_call(
        _gmlp_body, grid_spec=grid_spec,
        out_shape=jax.ShapeDtypeStruct((NP, H), jnp.float32),
    )(te, xs, Wg, Wu, Wd)


def _smlp_body(x_ref, wg_ref, wu_ref, wd_ref, y_ref, wg16, wu16, wd16):
    @pl.when(pl.program_id(0) == 0)
    def _():
        wg16[...] = wg_ref[...].astype(jnp.bfloat16)
        wu16[...] = wu_ref[...].astype(jnp.bfloat16)
        wd16[...] = wd_ref[...].astype(jnp.bfloat16)

    x = x_ref[...].astype(jnp.bfloat16)
    g = jnp.dot(x, wg16[...], preferred_element_type=jnp.float32)
    up = jnp.dot(x, wu16[...], preferred_element_type=jnp.float32)
    h = (_silu(g) * up).astype(jnp.bfloat16)
    y_ref[...] = jnp.dot(h, wd16[...], preferred_element_type=jnp.float32)


def _run_smlp(flat, Wg_s, Wu_s, Wd_s):
    return pl.pallas_call(
        _smlp_body,
        grid=(NSTEP,),
        in_specs=[pl.BlockSpec((TS, H), lambda i: (i, 0)),
                  pl.BlockSpec((H, ISH), lambda i: (0, 0)),
                  pl.BlockSpec((H, ISH), lambda i: (0, 0)),
                  pl.BlockSpec((ISH, H), lambda i: (0, 0))],
        out_specs=pl.BlockSpec((TS, H), lambda i: (i, 0)),
        out_shape=jax.ShapeDtypeStruct((T, H), jnp.float32),
        scratch_shapes=[pltpu.VMEM((H, ISH), jnp.bfloat16),
                        pltpu.VMEM((H, ISH), jnp.bfloat16),
                        pltpu.VMEM((ISH, H), jnp.bfloat16)],
    )(flat, Wg_s, Wu_s, Wd_s)


def _combine_body(sh_hbm, ys_hbm, d0_hbm, d1_hbm, w0_hbm, w1_hbm, out_hbm,
                  d0_v, d1_v, w0_v, w1_v, shv, g0v, g1v, ov,
                  ssh0, ssh1, sg00, sg01, sg10, sg11, swb0, swb1):
    wid = lax.axis_index("s") * 2 + lax.axis_index("c")
    base = wid * TPW
    ssh, sg0, sg1, swb = (ssh0, ssh1), (sg00, sg01), (sg10, sg11), (swb0, swb1)
    pltpu.sync_copy(d0_hbm.at[wid], d0_v)
    pltpu.sync_copy(d1_hbm.at[wid], d1_v)
    pltpu.sync_copy(w0_hbm.at[pl.ds(base, TPW)], w0_v.at[pl.ds(0, TPW)])
    pltpu.sync_copy(w1_hbm.at[pl.ds(base, TPW)], w1_v.at[pl.ds(0, TPW)])

    def _sh_cp(c, b):
        return pltpu.make_async_copy(
            sh_hbm.at[pl.ds(base + c * CCH, CCH)], shv.at[b], ssh[b])

    def _g0_cp(c, b):
        return pltpu.make_async_copy(ys_hbm.at[d0_v.at[c]], g0v.at[b], sg0[b])

    def _g1_cp(c, b):
        return pltpu.make_async_copy(ys_hbm.at[d1_v.at[c]], g1v.at[b], sg1[b])

    def _wb_cp(c, b):
        return pltpu.make_async_copy(
            ov.at[b], out_hbm.at[pl.ds(base + c * CCH, CCH)], swb[b])

    def _issue(c, b):
        _sh_cp(c, b).start()
        _g0_cp(c, b).start()
        _g1_cp(c, b).start()

    _issue(0, 0)

    def _pair(g, _):
        for b in range(2):
            c = g * 2 + b

            @pl.when(c + 1 < NCC)
            def _():
                _issue(c + 1, 1 - b)

            _sh_cp(c, b).wait()
            _g0_cp(c, b).wait()
            _g1_cp(c, b).wait()

            @pl.when(c >= 2)
            def _():
                _wb_cp(c - 2, b).wait()

            def _row(r, _2, b=b, c=c):
                a = w0_v[pl.ds(c * CCH + r, 16)][0]
                bb = w1_v[pl.ds(c * CCH + r, 16)][0]
                for j in range(H // 16):
                    sl = pl.ds(j * 16, 16)
                    ov[b, r, sl] = (shv[b, r, sl] + a * g0v[b, r, sl]
                                    + bb * g1v[b, r, sl])
                return 0

            lax.fori_loop(0, CCH, _row, 0)
            _wb_cp(c, b).start()
        return 0

    lax.fori_loop(0, NCC // 2, _pair, 0)
    _wb_cp(NCC - 2, 0).wait()
    _wb_cp(NCC - 1, 1).wait()


def _run_combine(sh, ys, d0r, d1r, w0f, w1f):
    mesh = plsc.VectorSubcoreMesh(core_axis_name="c", subcore_axis_name="s")
    scratch = [pltpu.VMEM((NCC, CCH), jnp.int32),
               pltpu.VMEM((NCC, CCH), jnp.int32),
               pltpu.VMEM((TPW + 16,), jnp.float32),
               pltpu.VMEM((TPW + 16,), jnp.float32),
               pltpu.VMEM((2, CCH, H), jnp.float32),
               pltpu.VMEM((2, CCH, H), jnp.float32),
               pltpu.VMEM((2, CCH, H), jnp.float32),
               pltpu.VMEM((2, CCH, H), jnp.float32)] + \
              [pltpu.SemaphoreType.DMA] * 8
    fn = pl.kernel(_combine_body,
                   out_type=jax.ShapeDtypeStruct((T, H), jnp.float32),
                   mesh=mesh, scratch_types=scratch,
                   compiler_params=pltpu.CompilerParams(needs_layout_passes=False))
    return fn(sh, ys, d0r, d1r, w0f, w1f)


def kernel(hidden_states, labels, cluster_centers, Wg, Wu, Wd, Wg_s, Wu_s, Wd_s):
    flat = hidden_states.reshape(T, H)
    uncond = (jnp.repeat(labels, S) == UNCOND).astype(jnp.float32)
    uncond = uncond.reshape(NSTEP, 1, TS)

    (e0, e1, w0, w1, r0, r1, counts,
     x16) = _run_router(flat, uncond, cluster_centers)
    counts_i = counts[:, 0].astype(jnp.int32)
    e0f, e1f = e0.reshape(T), e1.reshape(T)
    r0f, r1f = r0.reshape(T), r1.reshape(T)

    xp = lax.bitcast_convert_type(x16.reshape(T, H // 2, 2), jnp.int32)
    xs, d0, d1, te = _run_dispatch(counts_i, e0f, e1f, r0f, r1f, xp)
    xs16 = lax.bitcast_convert_type(xs, jnp.bfloat16).reshape(NP, H)
    ys = _run_gmlp(te, xs16, Wg, Wu, Wd)
    sh = _run_smlp(flat, Wg_s, Wu_s, Wd_s)

    out = _run_combine(sh, ys,
                       d0.reshape(NW, NCC, CCH), d1.reshape(NW, NCC, CCH),
                       w0.reshape(T), w1.reshape(T))
    return out.reshape(B, S, H)


# speedup vs baseline: 2.9882x; 2.9882x over previous
import functools

import jax
import jax.numpy as jnp
from jax import lax
from jax.experimental import pallas as pl
from jax.experimental.pallas import tpu as pltpu
from jax.experimental.pallas import tpu_sc as plsc

B, S, H = 4, 2048, 1024
T = B * S
NR = 64
NE = NR + 1
EP = 128
IM = 512
ISH = 1024
UNCOND = 1000
ECNT = 72

TS = 1024
NSTEP = T // TS

TILE = 128
TILE_SH = 7
NT_G = (T * 2 + NE * (TILE - 1) + TILE - 1) // TILE
NP = NT_G * TILE
TE_LEN = 224

NW = 32
TPW = T // NW
CH = 32
NCH = TPW // CH
CCH = 8
NCC = TPW // CCH


def _router_body(uncond_ref, x_ref, cc_ref,
                 e0_ref, e1_ref, w0_ref, w1_ref, r0_ref, r1_ref, counts_ref,
                 tri16):
    step = pl.program_id(0)
    x = x_ref[...]
    cc = cc_ref[...]
    xn = x * (1.0 / jnp.maximum(
        jnp.sqrt(jnp.sum(x * x, axis=1, keepdims=True)), 1e-12))
    cn = cc * (1.0 / jnp.maximum(
        jnp.sqrt(jnp.sum(cc * cc, axis=1, keepdims=True)), 1e-12))
    cos = lax.dot_general(cn, xn, (((1,), (1,)), ((), ())),
                          preferred_element_type=jnp.float32)
    mx = jnp.max(cos, axis=0, keepdims=True)
    pexp = jnp.exp(cos - mx)
    w = pexp / jnp.sum(pexp, axis=0, keepdims=True)

    eio = lax.broadcasted_iota(jnp.int32, (NR, TS), 0).astype(jnp.float32)
    m1 = jnp.max(w, axis=0, keepdims=True)
    a1 = jnp.min(jnp.where(w == m1, eio, jnp.float32(EP)), axis=0, keepdims=True)
    wm = jnp.where(eio == a1, -1.0, w)
    m2 = jnp.max(wm, axis=0, keepdims=True)
    a2 = jnp.min(jnp.where(wm == m2, eio, jnp.float32(EP)), axis=0, keepdims=True)

    is_u = uncond_ref[0] > 0.5
    e0 = jnp.where(is_u, jnp.float32(NE - 1), a1)
    e1 = jnp.where(is_u, jnp.float32(NE - 1), a2)
    w0 = jnp.where(is_u, 1.0, m1)
    w1 = jnp.where(is_u, 0.0, m2)

    ei72 = lax.broadcasted_iota(jnp.int32, (ECNT, TS), 0).astype(jnp.float32)
    oh0 = (ei72 == e0).astype(jnp.bfloat16)
    oh1 = (ei72 == e1).astype(jnp.bfloat16)

    @pl.when(step == 0)
    def _():
        tri16[...] = (lax.broadcasted_iota(jnp.int32, (TS, TS), 0) <
                      lax.broadcasted_iota(jnp.int32, (TS, TS), 1)
                      ).astype(jnp.bfloat16)
        counts_ref[...] = jnp.zeros_like(counts_ref)

    tri = tri16[...]
    p0 = lax.dot_general(oh0, tri, (((1,), (0,)), ((), ())),
                         preferred_element_type=jnp.float32)
    p1 = lax.dot_general(oh1, tri, (((1,), (0,)), ((), ())),
                         preferred_element_type=jnp.float32)
    oh0f = oh0.astype(jnp.float32)
    oh1f = oh1.astype(jnp.float32)
    c0 = jnp.sum(oh0f, axis=1, keepdims=True)
    c1 = jnp.sum(oh1f, axis=1, keepdims=True)

    run = counts_ref[0:ECNT, 0:1]
    r0 = jnp.sum(oh0f * (run + p0), axis=0, keepdims=True)
    r1 = jnp.sum(oh1f * (run + c0 + p1), axis=0, keepdims=True)
    counts_ref[0:ECNT, :] = jnp.broadcast_to(run + c0 + c1, (ECNT, EP))

    e0_ref[0] = e0.astype(jnp.int32)
    e1_ref[0] = e1.astype(jnp.int32)
    w0_ref[0] = w0
    w1_ref[0] = w1
    r0_ref[0] = r0.astype(jnp.int32)
    r1_ref[0] = r1.astype(jnp.int32)


def _run_router(flat, uncond, cc):
    i1 = jax.ShapeDtypeStruct((NSTEP, 1, TS), jnp.int32)
    f1 = jax.ShapeDtypeStruct((NSTEP, 1, TS), jnp.float32)
    out_shape = (i1, i1, f1, f1, i1, i1,
                 jax.ShapeDtypeStruct((EP, EP), jnp.float32))
    vec_spec = pl.BlockSpec((1, 1, TS), lambda i: (i, 0, 0))
    return pl.pallas_call(
        _router_body,
        grid=(NSTEP,),
        in_specs=[vec_spec,
                  pl.BlockSpec((TS, H), lambda i: (i, 0)),
                  pl.BlockSpec((NR, H), lambda i: (0, 0))],
        out_specs=(vec_spec, vec_spec, vec_spec, vec_spec, vec_spec, vec_spec,
                   pl.BlockSpec((EP, EP), lambda i: (0, 0))),
        out_shape=out_shape,
        scratch_shapes=[pltpu.VMEM((TS, TS), jnp.bfloat16)],
    )(uncond, flat, cc)


def _dispatch_body(counts_hbm, e0_hbm, e1_hbm, r0_hbm, r1_hbm, flat_hbm,
                   xs_hbm, d0_hbm, d1_hbm, te_hbm,
                   counts_v, pend_v, pstart_v, te_v, ev, rv, d0_v, d1_v,
                   rows_v, rsem, ssem0, ssem1):
    wid = lax.axis_index("s") * 2 + lax.axis_index("c")
    base = wid * TPW

    pltpu.sync_copy(counts_hbm, counts_v)
    carry = jnp.int32(0)
    for c in range(EP // 16):
        sl = pl.ds(c * 16, 16)
        cv = counts_v[sl]
        pad = ((cv + (TILE - 1)) >> TILE_SH) << TILE_SH
        cs = plsc.cumsum(pad) + carry
        pend_v[sl] = cs
        pstart_v[sl] = cs - pad
        carry = jnp.max(cs)

    for c in range(TE_LEN // 16):
        thresh = (lax.iota(jnp.int32, 16) + c * 16) * TILE

        def _cnt(e, acc):
            pe = pend_v[pl.ds(e, 16)][0]
            return acc + jnp.where(pe <= thresh, 1, 0)

        cnt = lax.fori_loop(0, NE, _cnt, jnp.zeros((16,), jnp.int32))
        te_v[pl.ds(c * 16, 16)] = jnp.minimum(cnt, NE - 1)
    te_v[pl.ds(208, 16)] = jnp.zeros((16,), jnp.int32) + (carry >> TILE_SH)

    @pl.when(wid == 0)
    def _():
        pltpu.sync_copy(te_v, te_hbm)

    for e_hbm, r_hbm, d_v, d_hbm in ((e0_hbm, r0_hbm, d0_v, d0_hbm),
                                     (e1_hbm, r1_hbm, d1_v, d1_hbm)):
        pltpu.sync_copy(e_hbm.at[pl.ds(base, TPW)], ev)
        pltpu.sync_copy(r_hbm.at[pl.ds(base, TPW)], rv)
        for c in range(TPW // 16):
            sl = pl.ds(c * 16, 16)
            ps = plsc.load_gather(pstart_v, [ev[sl]])
            d_v[c * 16 // CH, pl.ds(c * 16 % CH, 16)] = ps + rv[sl]
        pltpu.sync_copy(d_v, d_hbm.at[wid])

    ssem = (ssem0, ssem1)
    rd = [None] * NCH
    sc = [None] * (2 * NCH)

    def _read(j):
        return pltpu.async_copy(
            flat_hbm.at[pl.ds(base + j * CH, CH)], rows_v.at[j % 2], rsem)

    rd[0] = _read(0)
    for j in range(NCH):
        rd[j].wait()
        sc[2 * j] = pltpu.async_copy(
            rows_v.at[j % 2], xs_hbm.at[d0_v.at[j]], ssem[j % 2])
        sc[2 * j + 1] = pltpu.async_copy(
            rows_v.at[j % 2], xs_hbm.at[d1_v.at[j]], ssem[j % 2])
        if j + 1 < NCH:
            if j >= 1:
                sc[2 * (j - 1)].wait()
                sc[2 * (j - 1) + 1].wait()
            rd[j + 1] = _read(j + 1)
    sc[-2].wait()
    sc[-1].wait()


def _run_dispatch(counts_i, e0f, e1f, r0f, r1f, flat):
    mesh = plsc.VectorSubcoreMesh(core_axis_name="c", subcore_axis_name="s")
    out_type = (jax.ShapeDtypeStruct((NP, H), jnp.float32),
                jax.ShapeDtypeStruct((NW, NCH, CH), jnp.int32),
                jax.ShapeDtypeStruct((NW, NCH, CH), jnp.int32),
                jax.ShapeDtypeStruct((TE_LEN,), jnp.int32))
    scratch = [pltpu.VMEM((EP,), jnp.int32),
               pltpu.VMEM((EP,), jnp.int32),
               pltpu.VMEM((EP,), jnp.int32),
               pltpu.VMEM((TE_LEN,), jnp.int32),
               pltpu.VMEM((TPW,), jnp.int32),
               pltpu.VMEM((TPW,), jnp.int32),
               pltpu.VMEM((NCH, CH), jnp.int32),
               pltpu.VMEM((NCH, CH), jnp.int32),
               pltpu.VMEM((2, CH, H), jnp.float32),
               pltpu.SemaphoreType.DMA,
               pltpu.SemaphoreType.DMA,
               pltpu.SemaphoreType.DMA]
    fn = pl.kernel(_dispatch_body, out_type=out_type, mesh=mesh,
                   scratch_types=scratch,
                   compiler_params=pltpu.CompilerParams(needs_layout_passes=False))
    return fn(counts_i, e0f, e1f, r0f, r1f, flat)


def _silu(g):
    return g * (1.0 / (1.0 + jnp.exp(-g)))


def _gmlp_body(te_ref, x_ref, wg_ref, wu_ref, wd_ref, y_ref):
    @pl.when(pl.program_id(0) < te_ref[208])
    def _():
        x = x_ref[...].astype(jnp.bfloat16)
        wg = wg_ref[0].astype(jnp.bfloat16)
        wu = wu_ref[0].astype(jnp.bfloat16)
        wd = wd_ref[0].astype(jnp.bfloat16)
        g = jnp.dot(x, wg, preferred_element_type=jnp.float32)
        up = jnp.dot(x, wu, preferred_element_type=jnp.float32)
        h = (_silu(g) * up).astype(jnp.bfloat16)
        y_ref[...] = jnp.dot(h, wd, preferred_element_type=jnp.float32)


def _run_gmlp(te, xs, Wg, Wu, Wd):
    grid_spec = pltpu.PrefetchScalarGridSpec(
        num_scalar_prefetch=1,
        grid=(NT_G,),
        in_specs=[pl.BlockSpec((TILE, H),
                               lambda i, te: (jnp.minimum(i, te[208] - 1), 0)),
                  pl.BlockSpec((1, H, IM),
                               lambda i, te: (te[jnp.minimum(i, te[208] - 1)], 0, 0)),
                  pl.BlockSpec((1, H, IM),
                               lambda i, te: (te[jnp.minimum(i, te[208] - 1)], 0, 0)),
                  pl.BlockSpec((1, IM, H),
                               lambda i, te: (te[jnp.minimum(i, te[208] - 1)], 0, 0))],
        out_specs=pl.BlockSpec((TILE, H), lambda i, te: (i, 0)),
    )
    return pl.pallas_call(
        _gmlp_body, grid_spec=grid_spec,
        out_shape=jax.ShapeDtypeStruct((NP, H), jnp.float32),
    )(te, xs, Wg, Wu, Wd)


def _smlp_body(x_ref, wg_ref, wu_ref, wd_ref, y_ref, wg16, wu16, wd16):
    @pl.when(pl.program_id(0) == 0)
    def _():
        wg16[...] = wg_ref[...].astype(jnp.bfloat16)
        wu16[...] = wu_ref[...].astype(jnp.bfloat16)
        wd16[...] = wd_ref[...].astype(jnp.bfloat16)

    x = x_ref[...].astype(jnp.bfloat16)
    g = jnp.dot(x, wg16[...], preferred_element_type=jnp.float32)
    up = jnp.dot(x, wu16[...], preferred_element_type=jnp.float32)
    h = (_silu(g) * up).astype(jnp.bfloat16)
    y_ref[...] = jnp.dot(h, wd16[...], preferred_element_type=jnp.float32)


def _run_smlp(flat, Wg_s, Wu_s, Wd_s):
    return pl.pallas_call(
        _smlp_body,
        grid=(NSTEP,),
        in_specs=[pl.BlockSpec((TS, H), lambda i: (i, 0)),
                  pl.BlockSpec((H, ISH), lambda i: (0, 0)),
                  pl.BlockSpec((H, ISH), lambda i: (0, 0)),
                  pl.BlockSpec((ISH, H), lambda i: (0, 0))],
        out_specs=pl.BlockSpec((TS, H), lambda i: (i, 0)),
        out_shape=jax.ShapeDtypeStruct((T, H), jnp.float32),
        scratch_shapes=[pltpu.VMEM((H, ISH), jnp.bfloat16),
                        pltpu.VMEM((H, ISH), jnp.bfloat16),
                        pltpu.VMEM((ISH, H), jnp.bfloat16)],
    )(flat, Wg_s, Wu_s, Wd_s)


def _combine_body(sh_hbm, ys_hbm, d0_hbm, d1_hbm, w0_hbm, w1_hbm, out_hbm,
                  d0_v, d1_v, w0_v, w1_v, shv, g0v, g1v, ov,
                  ssh0, ssh1, sg00, sg01, sg10, sg11, swb0, swb1):
    wid = lax.axis_index("s") * 2 + lax.axis_index("c")
    base = wid * TPW
    ssh, sg0, sg1, swb = (ssh0, ssh1), (sg00, sg01), (sg10, sg11), (swb0, swb1)
    pltpu.sync_copy(d0_hbm.at[wid], d0_v)
    pltpu.sync_copy(d1_hbm.at[wid], d1_v)
    pltpu.sync_copy(w0_hbm.at[pl.ds(base, TPW)], w0_v.at[pl.ds(0, TPW)])
    pltpu.sync_copy(w1_hbm.at[pl.ds(base, TPW)], w1_v.at[pl.ds(0, TPW)])

    def _sh_cp(c, b):
        return pltpu.make_async_copy(
            sh_hbm.at[pl.ds(base + c * CCH, CCH)], shv.at[b], ssh[b])

    def _g0_cp(c, b):
        return pltpu.make_async_copy(ys_hbm.at[d0_v.at[c]], g0v.at[b], sg0[b])

    def _g1_cp(c, b):
        return pltpu.make_async_copy(ys_hbm.at[d1_v.at[c]], g1v.at[b], sg1[b])

    def _wb_cp(c, b):
        return pltpu.make_async_copy(
            ov.at[b], out_hbm.at[pl.ds(base + c * CCH, CCH)], swb[b])

    def _issue(c, b):
        _sh_cp(c, b).start()
        _g0_cp(c, b).start()
        _g1_cp(c, b).start()

    _issue(0, 0)

    def _pair(g, _):
        for b in range(2):
            c = g * 2 + b

            @pl.when(c + 1 < NCC)
            def _():
                _issue(c + 1, 1 - b)

            _sh_cp(c, b).wait()
            _g0_cp(c, b).wait()
            _g1_cp(c, b).wait()

            @pl.when(c >= 2)
            def _():
                _wb_cp(c - 2, b).wait()

            def _row(r, _2, b=b, c=c):
                a = w0_v[pl.ds(c * CCH + r, 16)][0]
                bb = w1_v[pl.ds(c * CCH + r, 16)][0]
                for j in range(H // 16):
                    sl = pl.ds(j * 16, 16)
                    ov[b, r, sl] = (shv[b, r, sl] + a * g0v[b, r, sl]
                                    + bb * g1v[b, r, sl])
                return 0

            lax.fori_loop(0, CCH, _row, 0)
            _wb_cp(c, b).start()
        return 0

    lax.fori_loop(0, NCC // 2, _pair, 0)
    _wb_cp(NCC - 2, 0).wait()
    _wb_cp(NCC - 1, 1).wait()


def _run_combine(sh, ys, d0r, d1r, w0f, w1f):
    mesh = plsc.VectorSubcoreMesh(core_axis_name="c", subcore_axis_name="s")
    scratch = [pltpu.VMEM((NCC, CCH), jnp.int32),
               pltpu.VMEM((NCC, CCH), jnp.int32),
               pltpu.VMEM((TPW + 16,), jnp.float32),
               pltpu.VMEM((TPW + 16,), jnp.float32),
               pltpu.VMEM((2, CCH, H), jnp.float32),
               pltpu.VMEM((2, CCH, H), jnp.float32),
               pltpu.VMEM((2, CCH, H), jnp.float32),
               pltpu.VMEM((2, CCH, H), jnp.float32)] + \
              [pltpu.SemaphoreType.DMA] * 8
    fn = pl.kernel(_combine_body,
                   out_type=jax.ShapeDtypeStruct((T, H), jnp.float32),
                   mesh=mesh, scratch_types=scratch,
                   compiler_params=pltpu.CompilerParams(needs_layout_passes=False))
    return fn(sh, ys, d0r, d1r, w0f, w1f)


def kernel(hidden_states, labels, cluster_centers, Wg, Wu, Wd, Wg_s, Wu_s, Wd_s):
    flat = hidden_states.reshape(T, H)
    uncond = (jnp.repeat(labels, S) == UNCOND).astype(jnp.float32)
    uncond = uncond.reshape(NSTEP, 1, TS)

    e0, e1, w0, w1, r0, r1, counts = _run_router(flat, uncond, cluster_centers)
    counts_i = counts[:, 0].astype(jnp.int32)
    e0f, e1f = e0.reshape(T), e1.reshape(T)
    r0f, r1f = r0.reshape(T), r1.reshape(T)

    xs, d0, d1, te = _run_dispatch(counts_i, e0f, e1f, r0f, r1f, flat)
    ys = _run_gmlp(te, xs, Wg, Wu, Wd)
    sh = _run_smlp(flat, Wg_s, Wu_s, Wd_s)

    out = _run_combine(sh, ys,
                       d0.reshape(NW, NCC, CCH), d1.reshape(NW, NCC, CCH),
                       w0.reshape(T), w1.reshape(T))
    return out.reshape(B, S, H)


# speedup vs baseline: 3.9789x; 1.3316x over previous
import functools

import jax
import jax.numpy as jnp
from jax import lax
from jax.experimental import pallas as pl
from jax.experimental.pallas import tpu as pltpu
from jax.experimental.pallas import tpu_sc as plsc

B, S, H = 4, 2048, 1024
T = B * S
NR = 64
NE = NR + 1
EP = 128
IM = 512
ISH = 1024
UNCOND = 1000
ECNT = 72

TS = 1024
NSTEP = T // TS

TILE = 512
TILE_SH = 9
NT_G = (T * 2 + NE * (TILE - 1) + TILE - 1) // TILE
NP = NT_G * TILE
TE_LEN = 128

NW = 32
TPW = T // NW
CH = 32
NCH = TPW // CH
CCH = 8
NCC = TPW // CCH


def _router_body(uncond_ref, x_ref, cc_ref,
                 e0_ref, e1_ref, w0_ref, w1_ref, r0_ref, r1_ref, counts_ref,
                 tri16):
    step = pl.program_id(0)
    x = x_ref[...]
    cc = cc_ref[...]
    xn = x * (1.0 / jnp.maximum(
        jnp.sqrt(jnp.sum(x * x, axis=1, keepdims=True)), 1e-12))
    cn = cc * (1.0 / jnp.maximum(
        jnp.sqrt(jnp.sum(cc * cc, axis=1, keepdims=True)), 1e-12))
    cos = lax.dot_general(cn, xn, (((1,), (1,)), ((), ())),
                          preferred_element_type=jnp.float32)
    mx = jnp.max(cos, axis=0, keepdims=True)
    pexp = jnp.exp(cos - mx)
    w = pexp / jnp.sum(pexp, axis=0, keepdims=True)

    eio = lax.broadcasted_iota(jnp.int32, (NR, TS), 0).astype(jnp.float32)
    m1 = jnp.max(w, axis=0, keepdims=True)
    a1 = jnp.min(jnp.where(w == m1, eio, jnp.float32(EP)), axis=0, keepdims=True)
    wm = jnp.where(eio == a1, -1.0, w)
    m2 = jnp.max(wm, axis=0, keepdims=True)
    a2 = jnp.min(jnp.where(wm == m2, eio, jnp.float32(EP)), axis=0, keepdims=True)

    is_u = uncond_ref[0] > 0.5
    e0 = jnp.where(is_u, jnp.float32(NE - 1), a1)
    e1 = jnp.where(is_u, jnp.float32(NE - 1), a2)
    w0 = jnp.where(is_u, 1.0, m1)
    w1 = jnp.where(is_u, 0.0, m2)

    ei72 = lax.broadcasted_iota(jnp.int32, (ECNT, TS), 0).astype(jnp.float32)
    oh0 = (ei72 == e0).astype(jnp.bfloat16)
    oh1 = (ei72 == e1).astype(jnp.bfloat16)

    @pl.when(step == 0)
    def _():
        tri16[...] = (lax.broadcasted_iota(jnp.int32, (TS, TS), 0) <
                      lax.broadcasted_iota(jnp.int32, (TS, TS), 1)
                      ).astype(jnp.bfloat16)
        counts_ref[...] = jnp.zeros_like(counts_ref)

    tri = tri16[...]
    p0 = lax.dot_general(oh0, tri, (((1,), (0,)), ((), ())),
                         preferred_element_type=jnp.float32)
    p1 = lax.dot_general(oh1, tri, (((1,), (0,)), ((), ())),
                         preferred_element_type=jnp.float32)
    oh0f = oh0.astype(jnp.float32)
    oh1f = oh1.astype(jnp.float32)
    c0 = jnp.sum(oh0f, axis=1, keepdims=True)
    c1 = jnp.sum(oh1f, axis=1, keepdims=True)

    run = counts_ref[0:ECNT, 0:1]
    r0 = jnp.sum(oh0f * (run + p0), axis=0, keepdims=True)
    r1 = jnp.sum(oh1f * (run + c0 + p1), axis=0, keepdims=True)
    counts_ref[0:ECNT, :] = jnp.broadcast_to(run + c0 + c1, (ECNT, EP))

    e0_ref[0] = e0.astype(jnp.int32)
    e1_ref[0] = e1.astype(jnp.int32)
    w0_ref[0] = w0
    w1_ref[0] = w1
    r0_ref[0] = r0.astype(jnp.int32)
    r1_ref[0] = r1.astype(jnp.int32)


def _run_router(flat, uncond, cc):
    i1 = jax.ShapeDtypeStruct((NSTEP, 1, TS), jnp.int32)
    f1 = jax.ShapeDtypeStruct((NSTEP, 1, TS), jnp.float32)
    out_shape = (i1, i1, f1, f1, i1, i1,
                 jax.ShapeDtypeStruct((EP, EP), jnp.float32))
    vec_spec = pl.BlockSpec((1, 1, TS), lambda i: (i, 0, 0))
    return pl.pallas_call(
        _router_body,
        grid=(NSTEP,),
        in_specs=[vec_spec,
                  pl.BlockSpec((TS, H), lambda i: (i, 0)),
                  pl.BlockSpec((NR, H), lambda i: (0, 0))],
        out_specs=(vec_spec, vec_spec, vec_spec, vec_spec, vec_spec, vec_spec,
                   pl.BlockSpec((EP, EP), lambda i: (0, 0))),
        out_shape=out_shape,
        scratch_shapes=[pltpu.VMEM((TS, TS), jnp.bfloat16)],
    )(uncond, flat, cc)


def _dispatch_body(counts_hbm, e0_hbm, e1_hbm, r0_hbm, r1_hbm, flat_hbm,
                   xs_hbm, d0_hbm, d1_hbm, te_hbm,
                   counts_v, pend_v, pstart_v, te_v, ev, rv, d0_v, d1_v,
                   rows_v, rsem, ssem0, ssem1):
    wid = lax.axis_index("s") * 2 + lax.axis_index("c")
    base = wid * TPW

    pltpu.sync_copy(counts_hbm, counts_v)
    carry = jnp.int32(0)
    for c in range(EP // 16):
        sl = pl.ds(c * 16, 16)
        cv = counts_v[sl]
        pad = ((cv + (TILE - 1)) >> TILE_SH) << TILE_SH
        cs = plsc.cumsum(pad) + carry
        pend_v[sl] = cs
        pstart_v[sl] = cs - pad
        carry = jnp.max(cs)

    for c in range(TE_LEN // 16):
        thresh = (lax.iota(jnp.int32, 16) + c * 16) * TILE

        def _cnt(e, acc):
            pe = pend_v[pl.ds(e, 16)][0]
            return acc + jnp.where(pe <= thresh, 1, 0)

        cnt = lax.fori_loop(0, NE, _cnt, jnp.zeros((16,), jnp.int32))
        te_v[pl.ds(c * 16, 16)] = jnp.minimum(cnt, NE - 1)
    te_v[pl.ds(112, 16)] = jnp.zeros((16,), jnp.int32) + (carry >> TILE_SH)

    @pl.when(wid == 0)
    def _():
        pltpu.sync_copy(te_v, te_hbm)

    for e_hbm, r_hbm, d_v, d_hbm in ((e0_hbm, r0_hbm, d0_v, d0_hbm),
                                     (e1_hbm, r1_hbm, d1_v, d1_hbm)):
        pltpu.sync_copy(e_hbm.at[pl.ds(base, TPW)], ev)
        pltpu.sync_copy(r_hbm.at[pl.ds(base, TPW)], rv)
        for c in range(TPW // 16):
            sl = pl.ds(c * 16, 16)
            ps = plsc.load_gather(pstart_v, [ev[sl]])
            d_v[c * 16 // CH, pl.ds(c * 16 % CH, 16)] = ps + rv[sl]
        pltpu.sync_copy(d_v, d_hbm.at[wid])

    ssem = (ssem0, ssem1)
    rd = [None] * NCH
    sc = [None] * (2 * NCH)

    def _read(j):
        return pltpu.async_copy(
            flat_hbm.at[pl.ds(base + j * CH, CH)], rows_v.at[j % 2], rsem)

    rd[0] = _read(0)
    for j in range(NCH):
        rd[j].wait()
        sc[2 * j] = pltpu.async_copy(
            rows_v.at[j % 2], xs_hbm.at[d0_v.at[j]], ssem[j % 2])
        sc[2 * j + 1] = pltpu.async_copy(
            rows_v.at[j % 2], xs_hbm.at[d1_v.at[j]], ssem[j % 2])
        if j + 1 < NCH:
            if j >= 1:
                sc[2 * (j - 1)].wait()
                sc[2 * (j - 1) + 1].wait()
            rd[j + 1] = _read(j + 1)
    sc[-2].wait()
    sc[-1].wait()


def _run_dispatch(counts_i, e0f, e1f, r0f, r1f, flat):
    mesh = plsc.VectorSubcoreMesh(core_axis_name="c", subcore_axis_name="s")
    out_type = (jax.ShapeDtypeStruct((NP, H), jnp.float32),
                jax.ShapeDtypeStruct((NW, NCH, CH), jnp.int32),
                jax.ShapeDtypeStruct((NW, NCH, CH), jnp.int32),
                jax.ShapeDtypeStruct((TE_LEN,), jnp.int32))
    scratch = [pltpu.VMEM((EP,), jnp.int32),
               pltpu.VMEM((EP,), jnp.int32),
               pltpu.VMEM((EP,), jnp.int32),
               pltpu.VMEM((TE_LEN,), jnp.int32),
               pltpu.VMEM((TPW,), jnp.int32),
               pltpu.VMEM((TPW,), jnp.int32),
               pltpu.VMEM((NCH, CH), jnp.int32),
               pltpu.VMEM((NCH, CH), jnp.int32),
               pltpu.VMEM((2, CH, H), jnp.float32),
               pltpu.SemaphoreType.DMA,
               pltpu.SemaphoreType.DMA,
               pltpu.SemaphoreType.DMA]
    fn = pl.kernel(_dispatch_body, out_type=out_type, mesh=mesh,
                   scratch_types=scratch,
                   compiler_params=pltpu.CompilerParams(needs_layout_passes=False))
    return fn(counts_i, e0f, e1f, r0f, r1f, flat)


def _silu(g):
    return g * (1.0 / (1.0 + jnp.exp(-g)))


def _gmlp_body(te_ref, x_ref, wg_ref, wu_ref, wd_ref, y_ref):
    @pl.when(pl.program_id(0) < te_ref[112])
    def _():
        x = x_ref[...].astype(jnp.bfloat16)
        wg = wg_ref[0].astype(jnp.bfloat16)
        wu = wu_ref[0].astype(jnp.bfloat16)
        wd = wd_ref[0].astype(jnp.bfloat16)
        g = jnp.dot(x, wg, preferred_element_type=jnp.float32)
        up = jnp.dot(x, wu, preferred_element_type=jnp.float32)
        h = (_silu(g) * up).astype(jnp.bfloat16)
        y = jnp.dot(h, wd, preferred_element_type=jnp.float32)
        ia = lax.bitcast_convert_type(y[:, :IM], jnp.int32)
        ib = lax.bitcast_convert_type(y[:, IM:], jnp.int32)
        ra = ((ia + jnp.int32(0x7FFF) + ((ia >> 16) & 1)) >> 16) & jnp.int32(0xFFFF)
        rb = (ib + jnp.int32(0x7FFF) + ((ib >> 16) & 1)) & jnp.int32(-65536)
        y_ref[...] = ra | rb


def _run_gmlp(te, xs, Wg, Wu, Wd):
    grid_spec = pltpu.PrefetchScalarGridSpec(
        num_scalar_prefetch=1,
        grid=(NT_G,),
        in_specs=[pl.BlockSpec((TILE, H),
                               lambda i, te: (jnp.minimum(i, te[112] - 1), 0)),
                  pl.BlockSpec((1, H, IM),
                               lambda i, te: (te[jnp.minimum(i, te[112] - 1)], 0, 0)),
                  pl.BlockSpec((1, H, IM),
                               lambda i, te: (te[jnp.minimum(i, te[112] - 1)], 0, 0)),
                  pl.BlockSpec((1, IM, H),
                               lambda i, te: (te[jnp.minimum(i, te[112] - 1)], 0, 0))],
        out_specs=pl.BlockSpec((TILE, IM), lambda i, te: (i, 0)),
    )
    return pl.pallas_call(
        _gmlp_body, grid_spec=grid_spec,
        out_shape=jax.ShapeDtypeStruct((NP, IM), jnp.int32),
    )(te, xs, Wg, Wu, Wd)


def _smlp_body(x_ref, wg_ref, wu_ref, wd_ref, y_ref, wg16, wu16, wd16):
    @pl.when(pl.program_id(0) == 0)
    def _():
        wg16[...] = wg_ref[...].astype(jnp.bfloat16)
        wu16[...] = wu_ref[...].astype(jnp.bfloat16)
        wd16[...] = wd_ref[...].astype(jnp.bfloat16)

    x = x_ref[...].astype(jnp.bfloat16)
    g = jnp.dot(x, wg16[...], preferred_element_type=jnp.float32)
    up = jnp.dot(x, wu16[...], preferred_element_type=jnp.float32)
    h = (_silu(g) * up).astype(jnp.bfloat16)
    y_ref[...] = jnp.dot(h, wd16[...], preferred_element_type=jnp.float32)


def _run_smlp(flat, Wg_s, Wu_s, Wd_s):
    return pl.pallas_call(
        _smlp_body,
        grid=(NSTEP,),
        in_specs=[pl.BlockSpec((TS, H), lambda i: (i, 0)),
                  pl.BlockSpec((H, ISH), lambda i: (0, 0)),
                  pl.BlockSpec((H, ISH), lambda i: (0, 0)),
                  pl.BlockSpec((ISH, H), lambda i: (0, 0))],
        out_specs=pl.BlockSpec((TS, H), lambda i: (i, 0)),
        out_shape=jax.ShapeDtypeStruct((T, H), jnp.float32),
        scratch_shapes=[pltpu.VMEM((H, ISH), jnp.bfloat16),
                        pltpu.VMEM((H, ISH), jnp.bfloat16),
                        pltpu.VMEM((ISH, H), jnp.bfloat16)],
    )(flat, Wg_s, Wu_s, Wd_s)


def _combine_body(sh_hbm, ys_hbm, d0_hbm, d1_hbm, w0_hbm, w1_hbm, out_hbm,
                  d0_v, d1_v, w0_v, w1_v, shv, g0v, g1v, ov,
                  ssh0, ssh1, sg00, sg01, sg10, sg11, swb0, swb1):
    wid = lax.axis_index("s") * 2 + lax.axis_index("c")
    base = wid * TPW
    ssh, sg0, sg1, swb = (ssh0, ssh1), (sg00, sg01), (sg10, sg11), (swb0, swb1)
    pltpu.sync_copy(d0_hbm.at[wid], d0_v)
    pltpu.sync_copy(d1_hbm.at[wid], d1_v)
    pltpu.sync_copy(w0_hbm.at[pl.ds(base, TPW)], w0_v.at[pl.ds(0, TPW)])
    pltpu.sync_copy(w1_hbm.at[pl.ds(base, TPW)], w1_v.at[pl.ds(0, TPW)])

    def _sh_cp(c, b):
        return pltpu.make_async_copy(
            sh_hbm.at[pl.ds(base + c * CCH, CCH)], shv.at[b], ssh[b])

    def _g0_cp(c, b):
        return pltpu.make_async_copy(ys_hbm.at[d0_v.at[c]], g0v.at[b], sg0[b])

    def _g1_cp(c, b):
        return pltpu.make_async_copy(ys_hbm.at[d1_v.at[c]], g1v.at[b], sg1[b])

    def _wb_cp(c, b):
        return pltpu.make_async_copy(
            ov.at[b], out_hbm.at[pl.ds(base + c * CCH, CCH)], swb[b])

    def _issue(c, b):
        _sh_cp(c, b).start()
        _g0_cp(c, b).start()
        _g1_cp(c, b).start()

    _issue(0, 0)

    def _pair(g, _):
        for b in range(2):
            c = g * 2 + b

            @pl.when(c + 1 < NCC)
            def _():
                _issue(c + 1, 1 - b)

            _sh_cp(c, b).wait()
            _g0_cp(c, b).wait()
            _g1_cp(c, b).wait()

            @pl.when(c >= 2)
            def _():
                _wb_cp(c - 2, b).wait()

            def _row(r, _2, b=b, c=c):
                a = w0_v[pl.ds(c * CCH + r, 16)][0]
                bb = w1_v[pl.ds(c * CCH + r, 16)][0]
                for j in range(IM // 16):
                    sl = pl.ds(j * 16, 16)
                    gi0 = g0v[b, r, sl]
                    gi1 = g1v[b, r, sl]
                    lo0 = plsc.bitcast(gi0 << 16, jnp.float32)
                    hi0 = plsc.bitcast(gi0 & jnp.int32(-65536), jnp.float32)
                    lo1 = plsc.bitcast(gi1 << 16, jnp.float32)
                    hi1 = plsc.bitcast(gi1 & jnp.int32(-65536), jnp.float32)
                    ov[b, r, sl] = shv[b, r, sl] + a * lo0 + bb * lo1
                    sl2 = pl.ds(IM + j * 16, 16)
                    ov[b, r, sl2] = shv[b, r, sl2] + a * hi0 + bb * hi1
                return 0

            lax.fori_loop(0, CCH, _row, 0)
            _wb_cp(c, b).start()
        return 0

    lax.fori_loop(0, NCC // 2, _pair, 0)
    _wb_cp(NCC - 2, 0).wait()
    _wb_cp(NCC - 1, 1).wait()


def _run_combine(sh, ys, d0r, d1r, w0f, w1f):
    mesh = plsc.VectorSubcoreMesh(core_axis_name="c", subcore_axis_name="s")
    scratch = [pltpu.VMEM((NCC, CCH), jnp.int32),
               pltpu.VMEM((NCC, CCH), jnp.int32),
               pltpu.VMEM((TPW + 16,), jnp.float32),
               pltpu.VMEM((TPW + 16,), jnp.float32),
               pltpu.VMEM((2, CCH, H), jnp.float32),
               pltpu.VMEM((2, CCH, IM), jnp.int32),
               pltpu.VMEM((2, CCH, IM), jnp.int32),
               pltpu.VMEM((2, CCH, H), jnp.float32)] + \
              [pltpu.SemaphoreType.DMA] * 8
    fn = pl.kernel(_combine_body,
                   out_type=jax.ShapeDtypeStruct((T, H), jnp.float32),
                   mesh=mesh, scratch_types=scratch,
                   compiler_params=pltpu.CompilerParams(needs_layout_passes=False))
    return fn(sh, ys, d0r, d1r, w0f, w1f)


def kernel(hidden_states, labels, cluster_centers, Wg, Wu, Wd, Wg_s, Wu_s, Wd_s):
    flat = hidden_states.reshape(T, H)
    uncond = (jnp.repeat(labels, S) == UNCOND).astype(jnp.float32)
    uncond = uncond.reshape(NSTEP, 1, TS)

    e0, e1, w0, w1, r0, r1, counts = _run_router(flat, uncond, cluster_centers)
    counts_i = counts[:, 0].astype(jnp.int32)
    e0f, e1f = e0.reshape(T), e1.reshape(T)
    r0f, r1f = r0.reshape(T), r1.reshape(T)

    xs, d0, d1, te = _run_dispatch(counts_i, e0f, e1f, r0f, r1f, flat)
    ys = _run_gmlp(te, xs, Wg, Wu, Wd)
    sh = _run_smlp(flat, Wg_s, Wu_s, Wd_s)

    out = _run_combine(sh, ys,
                       d0.reshape(NW, NCC, CCH), d1.reshape(NW, NCC, CCH),
                       w0.reshape(T), w1.reshape(T))
    return out.reshape(B, S, H)
